# Initial kernel scaffold; baseline (speedup 1.0000x reference)
import jax
import jax.numpy as jnp
from jax.experimental import pallas as pl


def kernel(q_data, target, q_embed_table, mem_key, mem_value_init, W_read, b_read, W_a, b_a, W_e, b_e, W_add, b_add, W_ih, W_hh, b_ih, b_hh, W_pred, b_pred):
    raise NotImplementedError("write your pallas kernel here")


# speedup vs baseline: 4.4742x; 4.4742x over previous
import functools

import jax
import jax.numpy as jnp
from jax import lax
from jax.experimental import pallas as pl
from jax.experimental.pallas import tpu as pltpu
from jax.experimental.pallas import tpu_sc as plsc

_B, _S = 64, 200
_QD = 128
_M = 64
_DV = 256
_FC = 110
_FCP = 128
_H = 64
_NW = 32
_ROWS = _B * _S
_RPW = _ROWS // _NW


def _sc_gather(table, idx):
    mesh = plsc.VectorSubcoreMesh(core_axis_name="c", subcore_axis_name="s")

    @functools.partial(
        pl.kernel,
        mesh=mesh,
        out_type=jax.ShapeDtypeStruct((_ROWS, _QD), jnp.float32),
        scratch_types=[
            pltpu.VMEM((_RPW,), jnp.int32),
            pltpu.VMEM((_RPW, _QD), jnp.float32),
            pltpu.SemaphoreType.DMA,
        ],
    )
    def k(table_hbm, idx_hbm, out_hbm, idx_v, rows_v, sem):
        wid = lax.axis_index("s") * 2 + lax.axis_index("c")
        base = wid * _RPW
        pltpu.sync_copy(idx_hbm.at[pl.ds(base, _RPW)], idx_v)
        pltpu.async_copy(table_hbm.at[idx_v], rows_v, sem).wait()
        pltpu.sync_copy(rows_v, out_hbm.at[pl.ds(base, _RPW)])

    return k(table, idx)


def _scan_body(qe_ref, tcol_ref, mkT_ref, vinit_ref,
               Wr1_ref, Wr2_ref, br_ref,
               WamT_ref, wat_ref, ba_ref,
               WeT_ref, be_ref, WaddT_ref, badd_ref,
               Wi1_ref, Wi2_ref, WhhT_ref, blstm_ref,
               wp_ref, bp_ref,
               out_ref, value_ref, hs_ref):
    value_ref[...] = jnp.broadcast_to(vinit_ref[...][None], (_B, _M, _DV))
    mkT = mkT_ref[...]
    Wr1 = Wr1_ref[...]
    Wr2 = Wr2_ref[...]
    br = br_ref[...]
    WamT = WamT_ref[...]
    wat = wat_ref[...]
    ba = ba_ref[...]
    WeT = WeT_ref[...]
    be = be_ref[...]
    WaddT = WaddT_ref[...]
    badd = badd_ref[...]
    Wi1 = Wi1_ref[...]
    Wi2 = Wi2_ref[...]
    WhhT = WhhT_ref[...]
    blstm = blstm_ref[...]

    def step(t, carry):
        h, c = carry
        q_t = qe_ref[t]
        tcol = tcol_ref[pl.ds(t * _B, _B), :]
        logits = jnp.dot(q_t, mkT, preferred_element_type=jnp.float32)
        mx = jnp.max(logits, axis=1, keepdims=True)
        ex = jnp.exp(logits - mx)
        cw = ex / jnp.sum(ex, axis=1, keepdims=True)
        value = value_ref[...]
        t1 = cw[:, :, None] * value
        read = jnp.sum(t1, axis=1)
        mastery = jnp.tanh(
            jnp.dot(read, Wr1, preferred_element_type=jnp.float32)
            + jnp.dot(q_t, Wr2, preferred_element_type=jnp.float32) + br)
        qa = jnp.tanh(
            jnp.dot(mastery, WamT, preferred_element_type=jnp.float32)
            + tcol * wat + ba)
        erase = jax.nn.sigmoid(
            jnp.dot(qa, WeT, preferred_element_type=jnp.float32) + be)
        addv = jnp.tanh(
            jnp.dot(qa, WaddT, preferred_element_type=jnp.float32) + badd)
        value_ref[...] = (value - t1 * erase[:, None, :]
                          + cw[:, :, None] * addv[:, None, :])
        gates = (jnp.dot(read, Wi1, preferred_element_type=jnp.float32)
                 + jnp.dot(q_t, Wi2, preferred_element_type=jnp.float32)
                 + jnp.dot(h, WhhT, preferred_element_type=jnp.float32)
                 + blstm)
        ig = jax.nn.sigmoid(gates[:, 0:_H])
        fg = jax.nn.sigmoid(gates[:, _H:2 * _H])
        gg = jnp.tanh(gates[:, 2 * _H:3 * _H])
        og = jax.nn.sigmoid(gates[:, 3 * _H:4 * _H])
        c = fg * c + ig * gg
        h = og * jnp.tanh(c)
        hs_ref[t] = h
        return h, c

    z = jnp.zeros((_B, _H), jnp.float32)
    lax.fori_loop(0, _S, step, (z, z))
    hs = hs_ref[...]
    p = jnp.sum(hs * wp_ref[...][None], axis=2) + bp_ref[...]
    out_ref[...] = jax.nn.sigmoid(p)


def _scan_tc(qe, tcol, mkT, vinit, Wr1, Wr2, br, WamT, wat, ba,
             WeT, be, WaddT, badd, Wi1, Wi2, WhhT, blstm, wp, bp):
    return pl.pallas_call(
        _scan_body,
        out_shape=jax.ShapeDtypeStruct((_S, _B), jnp.float32),
        scratch_shapes=[
            pltpu.VMEM((_B, _M, _DV), jnp.float32),
            pltpu.VMEM((_S, _B, _H), jnp.float32),
        ],
    )(qe, tcol, mkT, vinit, Wr1, Wr2, br, WamT, wat, ba,
      WeT, be, WaddT, badd, Wi1, Wi2, WhhT, blstm, wp, bp)


def kernel(q_data, target, q_embed_table, mem_key, mem_value_init,
           W_read, b_read, W_a, b_a, W_e, b_e, W_add, b_add,
           W_ih, W_hh, b_ih, b_hh, W_pred, b_pred):
    idx = q_data.T.reshape(-1).astype(jnp.int32)
    qe = _sc_gather(q_embed_table, idx).reshape(_S, _B, _QD)
    tcol = target.T.reshape(_S * _B, 1)

    WrT = W_read.T
    pad = ((0, 0), (0, _FCP - _FC))
    Wr1 = jnp.pad(WrT[:_DV], pad)
    Wr2 = jnp.pad(WrT[_DV:], pad)
    br = jnp.pad(b_read, (0, _FCP - _FC))[None]
    WamT = jnp.pad(W_a[:, :_FC].T, ((0, _FCP - _FC), (0, 0)))
    wat = W_a[:, _FC][None]
    ba = b_a[None]
    WeT = W_e.T
    be = b_e[None]
    WaddT = W_add.T
    badd = b_add[None]
    WiT = W_ih.T
    Wi1 = WiT[:_DV]
    Wi2 = WiT[_DV:]
    WhhT = W_hh.T
    blstm = (b_ih + b_hh)[None]
    wp = W_pred
    bp = b_pred.reshape(1, 1)

    out = _scan_tc(qe, tcol, mkT=mem_key.T, vinit=mem_value_init,
                   Wr1=Wr1, Wr2=Wr2, br=br, WamT=WamT, wat=wat, ba=ba,
                   WeT=WeT, be=be, WaddT=WaddT, badd=badd,
                   Wi1=Wi1, Wi2=Wi2, WhhT=WhhT, blstm=blstm, wp=wp, bp=bp)
    return out.T


# speedup vs baseline: 6.6604x; 1.4886x over previous
import functools

import jax
import jax.numpy as jnp
from jax import lax
from jax.experimental import pallas as pl
from jax.experimental.pallas import tpu as pltpu
from jax.experimental.pallas import tpu_sc as plsc

_B, _S = 64, 200
_QD = 128
_M = 64
_DV = 256
_FC = 110
_FCP = 128
_H = 64
_NW = 32
_ROWS = _B * _S
_RPW = _ROWS // _NW


def _sc_gather(table, idx):
    mesh = plsc.VectorSubcoreMesh(core_axis_name="c", subcore_axis_name="s")

    @functools.partial(
        pl.kernel,
        mesh=mesh,
        out_type=jax.ShapeDtypeStruct((_ROWS, _QD), jnp.float32),
        scratch_types=[
            pltpu.VMEM((_RPW,), jnp.int32),
            pltpu.VMEM((_RPW, _QD), jnp.float32),
            pltpu.SemaphoreType.DMA,
        ],
    )
    def k(table_hbm, idx_hbm, out_hbm, idx_v, rows_v, sem):
        wid = lax.axis_index("s") * 2 + lax.axis_index("c")
        base = wid * _RPW
        pltpu.sync_copy(idx_hbm.at[pl.ds(base, _RPW)], idx_v)
        pltpu.async_copy(table_hbm.at[idx_v], rows_v, sem).wait()
        pltpu.sync_copy(rows_v, out_hbm.at[pl.ds(base, _RPW)])

    return k(table, idx)


def _scan_body(qe_ref, tcol_ref, mkT_ref, vinit_ref,
               Wr1_ref, Wr2_ref, br_ref,
               WamT_ref, wat_ref, ba_ref,
               WeT_ref, be_ref, WaddT_ref, badd_ref,
               Wi1_ref, Wi2_ref, WhhT_ref, blstm_ref,
               wp_ref, bp_ref,
               out_ref, value_ref, hs_ref, cw_ref, gxm_ref, gxl_ref):
    mkT = mkT_ref[...]
    Wr2 = Wr2_ref[...]
    br = br_ref[...]
    Wi2 = Wi2_ref[...]
    blstm = blstm_ref[...]
    _CH = 1600

    def pro(i, _):
        sl = pl.ds(i * _CH, _CH)
        q = qe_ref[sl, :]
        logits = jnp.dot(q, mkT, preferred_element_type=jnp.float32)
        mx = jnp.max(logits, axis=1, keepdims=True)
        ex = jnp.exp(logits - mx)
        rs = 1.0 / jnp.sum(ex, axis=1, keepdims=True)
        cw_ref[sl, :] = ex * rs
        gxm_ref[sl, :] = jnp.dot(q, Wr2,
                                 preferred_element_type=jnp.float32) + br
        gxl_ref[sl, :] = jnp.dot(q, Wi2,
                                 preferred_element_type=jnp.float32) + blstm
        return 0

    lax.fori_loop(0, (_S * _B) // _CH, pro, 0)
    value_ref[...] = jnp.broadcast_to(vinit_ref[...][None], (_B, _M, _DV))

    Wr1 = Wr1_ref[...]
    WamT = WamT_ref[...]
    wat = wat_ref[...]
    ba = ba_ref[...]
    WeT = WeT_ref[...]
    be = be_ref[...]
    WaddT = WaddT_ref[...]
    badd = badd_ref[...]
    Wi1 = Wi1_ref[...]
    WhhT = WhhT_ref[...]

    read0 = jnp.sum(cw_ref[pl.ds(0, _B), :][:, :, None] * value_ref[...],
                    axis=1)

    def step(t, carry):
        read, h, c = carry
        mastery = jnp.tanh(
            jnp.dot(read, Wr1, preferred_element_type=jnp.float32)
            + gxm_ref[pl.ds(t * _B, _B), :])
        qa = jnp.tanh(
            jnp.dot(mastery, WamT, preferred_element_type=jnp.float32)
            + tcol_ref[pl.ds(t * _B, _B), :] * wat + ba)
        erase = jax.nn.sigmoid(
            jnp.dot(qa, WeT, preferred_element_type=jnp.float32) + be)
        addv = jnp.tanh(
            jnp.dot(qa, WaddT, preferred_element_type=jnp.float32) + badd)
        gates = (jnp.dot(read, Wi1, preferred_element_type=jnp.float32)
                 + jnp.dot(h, WhhT, preferred_element_type=jnp.float32)
                 + gxl_ref[pl.ds(t * _B, _B), :])
        ig = jax.nn.sigmoid(gates[:, 0:_H])
        fg = jax.nn.sigmoid(gates[:, _H:2 * _H])
        gg = jnp.tanh(gates[:, 2 * _H:3 * _H])
        og = jax.nn.sigmoid(gates[:, 3 * _H:4 * _H])
        c = fg * c + ig * gg
        h = og * jnp.tanh(c)
        hs_ref[t] = h
        cw3 = cw_ref[pl.ds(t * _B, _B), :][:, :, None]
        tn = jnp.minimum(t + 1, _S - 1)
        cwn = cw_ref[pl.ds(tn * _B, _B), :][:, :, None]
        value = value_ref[...]
        t1 = cw3 * value
        nv = value - t1 * erase[:, None, :] + cw3 * addv[:, None, :]
        value_ref[...] = nv
        read_n = jnp.sum(cwn * nv, axis=1)
        return read_n, h, c

    z = jnp.zeros((_B, _H), jnp.float32)
    lax.fori_loop(0, _S, step, (read0, z, z))
    hs = hs_ref[...]
    p = jnp.sum(hs * wp_ref[...][None], axis=2) + bp_ref[...]
    out_ref[...] = jax.nn.sigmoid(p)


def _scan_tc(qe, tcol, mkT, vinit, Wr1, Wr2, br, WamT, wat, ba,
             WeT, be, WaddT, badd, Wi1, Wi2, WhhT, blstm, wp, bp):
    return pl.pallas_call(
        _scan_body,
        out_shape=jax.ShapeDtypeStruct((_S, _B), jnp.float32),
        scratch_shapes=[
            pltpu.VMEM((_B, _M, _DV), jnp.float32),
            pltpu.VMEM((_S, _B, _H), jnp.float32),
            pltpu.VMEM((_S * _B, _M), jnp.float32),
            pltpu.VMEM((_S * _B, _FCP), jnp.float32),
            pltpu.VMEM((_S * _B, 4 * _H), jnp.float32),
        ],
    )(qe, tcol, mkT, vinit, Wr1, Wr2, br, WamT, wat, ba,
      WeT, be, WaddT, badd, Wi1, Wi2, WhhT, blstm, wp, bp)


def kernel(q_data, target, q_embed_table, mem_key, mem_value_init,
           W_read, b_read, W_a, b_a, W_e, b_e, W_add, b_add,
           W_ih, W_hh, b_ih, b_hh, W_pred, b_pred):
    idx = q_data.T.reshape(-1).astype(jnp.int32)
    qe = _sc_gather(q_embed_table, idx)
    tcol = target.T.reshape(_S * _B, 1)

    WrT = W_read.T
    pad = ((0, 0), (0, _FCP - _FC))
    Wr1 = jnp.pad(WrT[:_DV], pad)
    Wr2 = jnp.pad(WrT[_DV:], pad)
    br = jnp.pad(b_read, (0, _FCP - _FC))[None]
    WamT = jnp.pad(W_a[:, :_FC].T, ((0, _FCP - _FC), (0, 0)))
    wat = W_a[:, _FC][None]
    ba = b_a[None]
    WeT = W_e.T
    be = b_e[None]
    WaddT = W_add.T
    badd = b_add[None]
    WiT = W_ih.T
    Wi1 = WiT[:_DV]
    Wi2 = WiT[_DV:]
    WhhT = W_hh.T
    blstm = (b_ih + b_hh)[None]
    wp = W_pred
    bp = b_pred.reshape(1, 1)

    out = _scan_tc(qe, tcol, mkT=mem_key.T, vinit=mem_value_init,
                   Wr1=Wr1, Wr2=Wr2, br=br, WamT=WamT, wat=wat, ba=ba,
                   WeT=WeT, be=be, WaddT=WaddT, badd=badd,
                   Wi1=Wi1, Wi2=Wi2, WhhT=WhhT, blstm=blstm, wp=wp, bp=bp)
    return out.T


# speedup vs baseline: 6.6824x; 1.0033x over previous
import functools

import jax
import jax.numpy as jnp
from jax import lax
from jax.experimental import pallas as pl
from jax.experimental.pallas import tpu as pltpu
from jax.experimental.pallas import tpu_sc as plsc

_B, _S = 64, 200
_QD = 128
_M = 64
_DV = 256
_FC = 110
_FCP = 128
_H = 64
_NW = 32
_ROWS = _B * _S
_RPW = _ROWS // _NW


def _sc_gather(table, idx):
    mesh = plsc.VectorSubcoreMesh(core_axis_name="c", subcore_axis_name="s")

    @functools.partial(
        pl.kernel,
        mesh=mesh,
        out_type=jax.ShapeDtypeStruct((_ROWS, _QD), jnp.float32),
        scratch_types=[
            pltpu.VMEM((_RPW,), jnp.int32),
            pltpu.VMEM((_RPW, _QD), jnp.float32),
            pltpu.SemaphoreType.DMA,
        ],
    )
    def k(table_hbm, idx_hbm, out_hbm, idx_v, rows_v, sem):
        wid = lax.axis_index("s") * 2 + lax.axis_index("c")
        base = wid * _RPW
        pltpu.sync_copy(idx_hbm.at[pl.ds(base, _RPW)], idx_v)
        pltpu.async_copy(table_hbm.at[idx_v], rows_v, sem).wait()
        pltpu.sync_copy(rows_v, out_hbm.at[pl.ds(base, _RPW)])

    return k(table, idx)


def _scan_body(qe_ref, tcol_ref, mkT_ref, vinit_ref,
               Wr1_ref, Wr2_ref, br_ref,
               WamT_ref, wat_ref, ba_ref,
               WeT_ref, be_ref, WaddT_ref, badd_ref,
               Wi1_ref, Wi2_ref, WhhT_ref, blstm_ref,
               wp_ref, bp_ref,
               out_ref, value_ref, hs_ref, cw_ref, gxm_ref, gxl_ref):
    mkT = mkT_ref[...]
    Wr2 = Wr2_ref[...]
    br = br_ref[...]
    Wi2 = Wi2_ref[...]
    blstm = blstm_ref[...]
    _CH = 1600

    def pro(i, _):
        sl = pl.ds(i * _CH, _CH)
        q = qe_ref[sl, :]
        logits = jnp.dot(q, mkT, preferred_element_type=jnp.float32)
        mx = jnp.max(logits, axis=1, keepdims=True)
        ex = jnp.exp(logits - mx)
        rs = 1.0 / jnp.sum(ex, axis=1, keepdims=True)
        cw_ref[sl, :] = (ex * rs).astype(jnp.bfloat16)
        gxm_ref[sl, :] = jnp.dot(q, Wr2,
                                 preferred_element_type=jnp.float32) + br
        gxl_ref[sl, :] = jnp.dot(q, Wi2,
                                 preferred_element_type=jnp.float32) + blstm
        return 0

    lax.fori_loop(0, (_S * _B) // _CH, pro, 0)
    value_ref[...] = jnp.broadcast_to(
        vinit_ref[...].astype(jnp.bfloat16)[None], (_B, _M, _DV))

    Wr1 = Wr1_ref[...]
    WamT = WamT_ref[...]
    wat = wat_ref[...]
    ba = ba_ref[...]
    WeT = WeT_ref[...]
    be = be_ref[...]
    WaddT = WaddT_ref[...]
    badd = badd_ref[...]
    Wi1 = Wi1_ref[...]
    WhhT = WhhT_ref[...]

    read0 = jnp.sum(cw_ref[pl.ds(0, _B), :][:, :, None] * value_ref[...],
                    axis=1).astype(jnp.float32)

    def step(t, carry):
        read, h, c = carry
        mastery = jnp.tanh(
            jnp.dot(read, Wr1, preferred_element_type=jnp.float32)
            + gxm_ref[pl.ds(t * _B, _B), :])
        qa = jnp.tanh(
            jnp.dot(mastery, WamT, preferred_element_type=jnp.float32)
            + tcol_ref[pl.ds(t * _B, _B), :] * wat + ba)
        erase = jax.nn.sigmoid(
            jnp.dot(qa, WeT, preferred_element_type=jnp.float32) + be)
        addv = jnp.tanh(
            jnp.dot(qa, WaddT, preferred_element_type=jnp.float32) + badd)
        gates = (jnp.dot(read, Wi1, preferred_element_type=jnp.float32)
                 + jnp.dot(h, WhhT, preferred_element_type=jnp.float32)
                 + gxl_ref[pl.ds(t * _B, _B), :])
        ig = jax.nn.sigmoid(gates[:, 0:_H])
        fg = jax.nn.sigmoid(gates[:, _H:2 * _H])
        gg = jnp.tanh(gates[:, 2 * _H:3 * _H])
        og = jax.nn.sigmoid(gates[:, 3 * _H:4 * _H])
        c = fg * c + ig * gg
        h = og * jnp.tanh(c)
        hs_ref[t] = h
        cw3 = cw_ref[pl.ds(t * _B, _B), :][:, :, None]
        tn = jnp.minimum(t + 1, _S - 1)
        cwn = cw_ref[pl.ds(tn * _B, _B), :][:, :, None]
        eb = erase.astype(jnp.bfloat16)
        ab = addv.astype(jnp.bfloat16)
        value = value_ref[...]
        t1 = cw3 * value
        nv = value - t1 * eb[:, None, :] + cw3 * ab[:, None, :]
        value_ref[...] = nv
        read_n = jnp.sum(cwn * nv, axis=1).astype(jnp.float32)
        return read_n, h, c

    z = jnp.zeros((_B, _H), jnp.float32)
    lax.fori_loop(0, _S, step, (read0, z, z))
    hs = hs_ref[...]
    p = jnp.sum(hs * wp_ref[...][None], axis=2) + bp_ref[...]
    out_ref[...] = jax.nn.sigmoid(p)


def _scan_tc(qe, tcol, mkT, vinit, Wr1, Wr2, br, WamT, wat, ba,
             WeT, be, WaddT, badd, Wi1, Wi2, WhhT, blstm, wp, bp):
    return pl.pallas_call(
        _scan_body,
        out_shape=jax.ShapeDtypeStruct((_S, _B), jnp.float32),
        scratch_shapes=[
            pltpu.VMEM((_B, _M, _DV), jnp.bfloat16),
            pltpu.VMEM((_S, _B, _H), jnp.float32),
            pltpu.VMEM((_S * _B, _M), jnp.bfloat16),
            pltpu.VMEM((_S * _B, _FCP), jnp.float32),
            pltpu.VMEM((_S * _B, 4 * _H), jnp.float32),
        ],
    )(qe, tcol, mkT, vinit, Wr1, Wr2, br, WamT, wat, ba,
      WeT, be, WaddT, badd, Wi1, Wi2, WhhT, blstm, wp, bp)


def kernel(q_data, target, q_embed_table, mem_key, mem_value_init,
           W_read, b_read, W_a, b_a, W_e, b_e, W_add, b_add,
           W_ih, W_hh, b_ih, b_hh, W_pred, b_pred):
    idx = q_data.T.reshape(-1).astype(jnp.int32)
    qe = _sc_gather(q_embed_table, idx)
    tcol = target.T.reshape(_S * _B, 1)

    WrT = W_read.T
    pad = ((0, 0), (0, _FCP - _FC))
    Wr1 = jnp.pad(WrT[:_DV], pad)
    Wr2 = jnp.pad(WrT[_DV:], pad)
    br = jnp.pad(b_read, (0, _FCP - _FC))[None]
    WamT = jnp.pad(W_a[:, :_FC].T, ((0, _FCP - _FC), (0, 0)))
    wat = W_a[:, _FC][None]
    ba = b_a[None]
    WeT = W_e.T
    be = b_e[None]
    WaddT = W_add.T
    badd = b_add[None]
    WiT = W_ih.T
    Wi1 = WiT[:_DV]
    Wi2 = WiT[_DV:]
    WhhT = W_hh.T
    blstm = (b_ih + b_hh)[None]
    wp = W_pred
    bp = b_pred.reshape(1, 1)

    out = _scan_tc(qe, tcol, mkT=mem_key.T, vinit=mem_value_init,
                   Wr1=Wr1, Wr2=Wr2, br=br, WamT=WamT, wat=wat, ba=ba,
                   WeT=WeT, be=be, WaddT=WaddT, badd=badd,
                   Wi1=Wi1, Wi2=Wi2, WhhT=WhhT, blstm=blstm, wp=wp, bp=bp)
    return out.T


# speedup vs baseline: 7.6850x; 1.1500x over previous
import functools

import jax
import jax.numpy as jnp
from jax import lax
from jax.experimental import pallas as pl
from jax.experimental.pallas import tpu as pltpu
from jax.experimental.pallas import tpu_sc as plsc

_B, _S = 64, 200
_QD = 128
_M = 64
_DV = 256
_FC = 110
_FCP = 128
_H = 64
_NW = 32
_ROWS = _B * _S
_RPW = _ROWS // _NW


def _sc_gather(table, idx):
    mesh = plsc.VectorSubcoreMesh(core_axis_name="c", subcore_axis_name="s")

    @functools.partial(
        pl.kernel,
        mesh=mesh,
        out_type=jax.ShapeDtypeStruct((_ROWS, _QD), jnp.float32),
        scratch_types=[
            pltpu.VMEM((_RPW,), jnp.int32),
            pltpu.VMEM((_RPW, _QD), jnp.float32),
            pltpu.SemaphoreType.DMA,
        ],
    )
    def k(table_hbm, idx_hbm, out_hbm, idx_v, rows_v, sem):
        wid = lax.axis_index("s") * 2 + lax.axis_index("c")
        base = wid * _RPW
        pltpu.sync_copy(idx_hbm.at[pl.ds(base, _RPW)], idx_v)
        pltpu.async_copy(table_hbm.at[idx_v], rows_v, sem).wait()
        pltpu.sync_copy(rows_v, out_hbm.at[pl.ds(base, _RPW)])

    return k(table, idx)


def _scan_body(qe_ref, tcol_ref, mkT_ref, vinit_ref,
               Wr1_ref, Wr2_ref, br_ref,
               WamT_ref, wat_ref, ba_ref,
               WeT_ref, be_ref, WaddT_ref, badd_ref,
               Wi1_ref, Wi2_ref, WhhT_ref, blstm_ref,
               wp_ref, bp_ref,
               out_ref, value_ref, hs_ref, cwT_ref, gxm_ref, gxl_ref):
    mkT = mkT_ref[...]
    Wr2 = Wr2_ref[...]
    br = br_ref[...]
    Wi2 = Wi2_ref[...]
    blstm = blstm_ref[...]
    _CH = 1600

    def pro(i, _):
        sl = pl.ds(i * _CH, _CH)
        q = qe_ref[sl, :]
        logits = jnp.dot(q, mkT, preferred_element_type=jnp.float32)
        mx = jnp.max(logits, axis=1, keepdims=True)
        ex = jnp.exp(logits - mx)
        rs = 1.0 / jnp.sum(ex, axis=1, keepdims=True)
        cw = (ex * rs).astype(jnp.bfloat16)
        cwT_ref[sl, :] = jnp.swapaxes(
            cw.reshape(_CH // _B, _B, _M), 1, 2).reshape(_CH, _B)
        gxm_ref[sl, :] = jnp.dot(q, Wr2,
                                 preferred_element_type=jnp.float32) + br
        gxl_ref[sl, :] = jnp.dot(q, Wi2,
                                 preferred_element_type=jnp.float32) + blstm
        return 0

    lax.fori_loop(0, (_S * _B) // _CH, pro, 0)
    value_ref[...] = jnp.broadcast_to(
        vinit_ref[...].astype(jnp.bfloat16)[:, None, :], (_M, _B, _DV))

    Wr1 = Wr1_ref[...]
    WamT = WamT_ref[...]
    wat = wat_ref[...]
    ba = ba_ref[...]
    WeT = WeT_ref[...]
    be = be_ref[...]
    WaddT = WaddT_ref[...]
    badd = badd_ref[...]
    Wi1 = Wi1_ref[...]
    WhhT = WhhT_ref[...]

    read0 = jnp.sum(cwT_ref[pl.ds(0, _M), :][:, :, None] * value_ref[...],
                    axis=0).astype(jnp.float32)

    def step(t, carry):
        read, h, c = carry
        mastery = jnp.tanh(
            jnp.dot(read, Wr1, preferred_element_type=jnp.float32)
            + gxm_ref[pl.ds(t * _B, _B), :])
        qa = jnp.tanh(
            jnp.dot(mastery, WamT, preferred_element_type=jnp.float32)
            + tcol_ref[pl.ds(t * _B, _B), :] * wat + ba)
        erase = jax.nn.sigmoid(
            jnp.dot(qa, WeT, preferred_element_type=jnp.float32) + be)
        addv = jnp.tanh(
            jnp.dot(qa, WaddT, preferred_element_type=jnp.float32) + badd)
        gates = (jnp.dot(read, Wi1, preferred_element_type=jnp.float32)
                 + jnp.dot(h, WhhT, preferred_element_type=jnp.float32)
                 + gxl_ref[pl.ds(t * _B, _B), :])
        ig = jax.nn.sigmoid(gates[:, 0:_H])
        fg = jax.nn.sigmoid(gates[:, _H:2 * _H])
        gg = jnp.tanh(gates[:, 2 * _H:3 * _H])
        og = jax.nn.sigmoid(gates[:, 3 * _H:4 * _H])
        c = fg * c + ig * gg
        h = og * jnp.tanh(c)
        hs_ref[t] = h
        cw3 = cwT_ref[pl.ds(t * _M, _M), :][:, :, None]
        tn = jnp.minimum(t + 1, _S - 1)
        cwn = cwT_ref[pl.ds(tn * _M, _M), :][:, :, None]
        eb = erase.astype(jnp.bfloat16)
        ab = addv.astype(jnp.bfloat16)
        value = value_ref[...]
        t1 = cw3 * value
        nv = value - t1 * eb[None, :, :] + cw3 * ab[None, :, :]
        value_ref[...] = nv
        read_n = jnp.sum(cwn * nv, axis=0).astype(jnp.float32)
        return read_n, h, c

    z = jnp.zeros((_B, _H), jnp.float32)
    lax.fori_loop(0, _S, step, (read0, z, z))
    hs = hs_ref[...]
    p = jnp.sum(hs * wp_ref[...][None], axis=2) + bp_ref[...]
    out_ref[...] = jax.nn.sigmoid(p)


def _scan_tc(qe, tcol, mkT, vinit, Wr1, Wr2, br, WamT, wat, ba,
             WeT, be, WaddT, badd, Wi1, Wi2, WhhT, blstm, wp, bp):
    return pl.pallas_call(
        _scan_body,
        out_shape=jax.ShapeDtypeStruct((_S, _B), jnp.float32),
        scratch_shapes=[
            pltpu.VMEM((_M, _B, _DV), jnp.bfloat16),
            pltpu.VMEM((_S, _B, _H), jnp.float32),
            pltpu.VMEM((_S * _B, _M), jnp.bfloat16),
            pltpu.VMEM((_S * _B, _FCP), jnp.float32),
            pltpu.VMEM((_S * _B, 4 * _H), jnp.float32),
        ],
    )(qe, tcol, mkT, vinit, Wr1, Wr2, br, WamT, wat, ba,
      WeT, be, WaddT, badd, Wi1, Wi2, WhhT, blstm, wp, bp)


def kernel(q_data, target, q_embed_table, mem_key, mem_value_init,
           W_read, b_read, W_a, b_a, W_e, b_e, W_add, b_add,
           W_ih, W_hh, b_ih, b_hh, W_pred, b_pred):
    idx = q_data.T.reshape(-1).astype(jnp.int32)
    qe = _sc_gather(q_embed_table, idx)
    tcol = target.T.reshape(_S * _B, 1)

    WrT = W_read.T
    pad = ((0, 0), (0, _FCP - _FC))
    Wr1 = jnp.pad(WrT[:_DV], pad)
    Wr2 = jnp.pad(WrT[_DV:], pad)
    br = jnp.pad(b_read, (0, _FCP - _FC))[None]
    WamT = jnp.pad(W_a[:, :_FC].T, ((0, _FCP - _FC), (0, 0)))
    wat = W_a[:, _FC][None]
    ba = b_a[None]
    WeT = W_e.T
    be = b_e[None]
    WaddT = W_add.T
    badd = b_add[None]
    WiT = W_ih.T
    Wi1 = WiT[:_DV]
    Wi2 = WiT[_DV:]
    WhhT = W_hh.T
    blstm = (b_ih + b_hh)[None]
    wp = W_pred
    bp = b_pred.reshape(1, 1)

    out = _scan_tc(qe, tcol, mkT=mem_key.T, vinit=mem_value_init,
                   Wr1=Wr1, Wr2=Wr2, br=br, WamT=WamT, wat=wat, ba=ba,
                   WeT=WeT, be=be, WaddT=WaddT, badd=badd,
                   Wi1=Wi1, Wi2=Wi2, WhhT=WhhT, blstm=blstm, wp=wp, bp=bp)
    return out.T


# speedup vs baseline: 9.4763x; 1.2331x over previous
import functools

import jax
import jax.numpy as jnp
from jax import lax
from jax.experimental import pallas as pl
from jax.experimental.pallas import tpu as pltpu
from jax.experimental.pallas import tpu_sc as plsc

_B, _S = 64, 200
_QD = 128
_M = 64
_DV = 256
_FC = 110
_FCP = 128
_H = 64
_NW = 32
_ROWS = _B * _S
_RPW = _ROWS // _NW


def _sc_gather(table, idx):
    mesh = plsc.VectorSubcoreMesh(core_axis_name="c", subcore_axis_name="s")

    @functools.partial(
        pl.kernel,
        mesh=mesh,
        out_type=jax.ShapeDtypeStruct((_ROWS, _QD), jnp.float32),
        scratch_types=[
            pltpu.VMEM((_RPW,), jnp.int32),
            pltpu.VMEM((_RPW, _QD), jnp.float32),
            pltpu.SemaphoreType.DMA,
        ],
    )
    def k(table_hbm, idx_hbm, out_hbm, idx_v, rows_v, sem):
        wid = lax.axis_index("s") * 2 + lax.axis_index("c")
        base = wid * _RPW
        pltpu.sync_copy(idx_hbm.at[pl.ds(base, _RPW)], idx_v)
        pltpu.async_copy(table_hbm.at[idx_v], rows_v, sem).wait()
        pltpu.sync_copy(rows_v, out_hbm.at[pl.ds(base, _RPW)])

    return k(table, idx)


def _scan_body(qe_ref, tcol_ref, mkT_ref, vinit_ref,
               Wr1_ref, Wr2_ref, br_ref,
               WamT_ref, wat_ref, ba_ref,
               WeT_ref, be_ref, WaddT_ref, badd_ref,
               Wi1_ref, Wi2_ref, WhhT_ref, blstm_ref,
               wp_ref, bp_ref,
               out_ref, value_ref, hs_ref, cwT_ref, gxm_ref, gxl_ref, t1_ref):
    mkT = mkT_ref[...]
    Wr2 = Wr2_ref[...]
    br = br_ref[...]
    Wi2 = Wi2_ref[...]
    blstm = blstm_ref[...]
    _CH = 1600

    def pro(i, _):
        sl = pl.ds(i * _CH, _CH)
        q = qe_ref[sl, :]
        logits = jnp.dot(q, mkT, preferred_element_type=jnp.float32)
        mx = jnp.max(logits, axis=1, keepdims=True)
        ex = jnp.exp(logits - mx)
        rs = 1.0 / jnp.sum(ex, axis=1, keepdims=True)
        cw = (ex * rs).astype(jnp.bfloat16)
        cwT_ref[sl, :] = jnp.swapaxes(
            cw.reshape(_CH // _B, _B, _M), 1, 2).reshape(_CH, _B)
        gxm_ref[sl, :] = jnp.dot(q, Wr2,
                                 preferred_element_type=jnp.float32) + br
        gxl_ref[sl, :] = jnp.dot(q, Wi2,
                                 preferred_element_type=jnp.float32) + blstm
        return 0

    lax.fori_loop(0, (_S * _B) // _CH, pro, 0)
    value_ref[...] = jnp.broadcast_to(
        vinit_ref[...].astype(jnp.bfloat16)[:, None, :], (_M, _B, _DV))

    Wr1 = Wr1_ref[...]
    WamT = WamT_ref[...]
    wat = wat_ref[...]
    ba = ba_ref[...]
    WeT = WeT_ref[...]
    be = be_ref[...]
    WaddT = WaddT_ref[...]
    badd = badd_ref[...]
    Wi1 = Wi1_ref[...]
    WhhT = WhhT_ref[...]

    def _treesum(prod):
        p4 = prod.reshape(_M // 4, 4, _B, _DV)
        sb = (p4[:, 0] + p4[:, 1]) + (p4[:, 2] + p4[:, 3])
        return jnp.sum(sb.astype(jnp.float32), axis=0)

    t1_ref[...] = cwT_ref[pl.ds(0, _M), :][:, :, None] * value_ref[...]
    read0 = _treesum(t1_ref[...])

    def step(t, carry):
        read, h, c = carry
        mastery = jnp.tanh(
            jnp.dot(read, Wr1, preferred_element_type=jnp.float32)
            + gxm_ref[pl.ds(t * _B, _B), :])
        qa = jnp.tanh(
            jnp.dot(mastery, WamT, preferred_element_type=jnp.float32)
            + tcol_ref[pl.ds(t * _B, _B), :] * wat + ba)
        erase = jax.nn.sigmoid(
            jnp.dot(qa, WeT, preferred_element_type=jnp.float32) + be)
        addv = jnp.tanh(
            jnp.dot(qa, WaddT, preferred_element_type=jnp.float32) + badd)
        gates = (jnp.dot(read, Wi1, preferred_element_type=jnp.float32)
                 + jnp.dot(h, WhhT, preferred_element_type=jnp.float32)
                 + gxl_ref[pl.ds(t * _B, _B), :])
        ig = jax.nn.sigmoid(gates[:, 0:_H])
        fg = jax.nn.sigmoid(gates[:, _H:2 * _H])
        gg = jnp.tanh(gates[:, 2 * _H:3 * _H])
        og = jax.nn.sigmoid(gates[:, 3 * _H:4 * _H])
        c = fg * c + ig * gg
        h = og * jnp.tanh(c)
        hs_ref[t] = h
        cw3 = cwT_ref[pl.ds(t * _M, _M), :][:, :, None]
        tn = jnp.minimum(t + 1, _S - 1)
        cwn = cwT_ref[pl.ds(tn * _M, _M), :][:, :, None]
        eb = erase.astype(jnp.bfloat16)
        ab = addv.astype(jnp.bfloat16)
        value = value_ref[...]
        t1 = t1_ref[...]
        nv = value - t1 * eb[None, :, :] + cw3 * ab[None, :, :]
        value_ref[...] = nv
        prod = cwn * nv
        t1_ref[...] = prod
        read_n = _treesum(prod)
        return read_n, h, c

    z = jnp.zeros((_B, _H), jnp.float32)
    lax.fori_loop(0, _S, step, (read0, z, z))
    hs = hs_ref[...]
    p = jnp.sum(hs * wp_ref[...][None], axis=2) + bp_ref[...]
    out_ref[...] = jax.nn.sigmoid(p)


def _scan_tc(qe, tcol, mkT, vinit, Wr1, Wr2, br, WamT, wat, ba,
             WeT, be, WaddT, badd, Wi1, Wi2, WhhT, blstm, wp, bp):
    return pl.pallas_call(
        _scan_body,
        out_shape=jax.ShapeDtypeStruct((_S, _B), jnp.float32),
        scratch_shapes=[
            pltpu.VMEM((_M, _B, _DV), jnp.bfloat16),
            pltpu.VMEM((_S, _B, _H), jnp.float32),
            pltpu.VMEM((_S * _B, _M), jnp.bfloat16),
            pltpu.VMEM((_S * _B, _FCP), jnp.float32),
            pltpu.VMEM((_S * _B, 4 * _H), jnp.float32),
            pltpu.VMEM((_M, _B, _DV), jnp.bfloat16),
        ],
    )(qe, tcol, mkT, vinit, Wr1, Wr2, br, WamT, wat, ba,
      WeT, be, WaddT, badd, Wi1, Wi2, WhhT, blstm, wp, bp)


def kernel(q_data, target, q_embed_table, mem_key, mem_value_init,
           W_read, b_read, W_a, b_a, W_e, b_e, W_add, b_add,
           W_ih, W_hh, b_ih, b_hh, W_pred, b_pred):
    idx = q_data.T.reshape(-1).astype(jnp.int32)
    qe = _sc_gather(q_embed_table, idx)
    tcol = target.T.reshape(_S * _B, 1)

    WrT = W_read.T
    pad = ((0, 0), (0, _FCP - _FC))
    Wr1 = jnp.pad(WrT[:_DV], pad)
    Wr2 = jnp.pad(WrT[_DV:], pad)
    br = jnp.pad(b_read, (0, _FCP - _FC))[None]
    WamT = jnp.pad(W_a[:, :_FC].T, ((0, _FCP - _FC), (0, 0)))
    wat = W_a[:, _FC][None]
    ba = b_a[None]
    WeT = W_e.T
    be = b_e[None]
    WaddT = W_add.T
    badd = b_add[None]
    WiT = W_ih.T
    Wi1 = WiT[:_DV]
    Wi2 = WiT[_DV:]
    WhhT = W_hh.T
    blstm = (b_ih + b_hh)[None]
    wp = W_pred
    bp = b_pred.reshape(1, 1)

    out = _scan_tc(qe, tcol, mkT=mem_key.T, vinit=mem_value_init,
                   Wr1=Wr1, Wr2=Wr2, br=br, WamT=WamT, wat=wat, ba=ba,
                   WeT=WeT, be=be, WaddT=WaddT, badd=badd,
                   Wi1=Wi1, Wi2=Wi2, WhhT=WhhT, blstm=blstm, wp=wp, bp=bp)
    return out.T


# speedup vs baseline: 9.5963x; 1.0127x over previous
import functools

import jax
import jax.numpy as jnp
from jax import lax
from jax.experimental import pallas as pl
from jax.experimental.pallas import tpu as pltpu
from jax.experimental.pallas import tpu_sc as plsc

_B, _S = 64, 200
_QD = 128
_M = 64
_DV = 256
_FC = 110
_FCP = 128
_H = 64
_NW = 32
_ROWS = _B * _S
_RPW = _ROWS // _NW


def _sc_gather(table, idx):
    mesh = plsc.VectorSubcoreMesh(core_axis_name="c", subcore_axis_name="s")

    @functools.partial(
        pl.kernel,
        mesh=mesh,
        out_type=jax.ShapeDtypeStruct((_ROWS, _QD), jnp.float32),
        scratch_types=[
            pltpu.VMEM((_RPW,), jnp.int32),
            pltpu.VMEM((_RPW, _QD), jnp.float32),
            pltpu.SemaphoreType.DMA,
        ],
    )
    def k(table_hbm, idx_hbm, out_hbm, idx_v, rows_v, sem):
        wid = lax.axis_index("s") * 2 + lax.axis_index("c")
        base = wid * _RPW
        pltpu.sync_copy(idx_hbm.at[pl.ds(base, _RPW)], idx_v)
        pltpu.async_copy(table_hbm.at[idx_v], rows_v, sem).wait()
        pltpu.sync_copy(rows_v, out_hbm.at[pl.ds(base, _RPW)])

    return k(table, idx)


def _scan_body(qe_ref, tcol_ref, mkT_ref, vinit_ref,
               Wr1_ref, Wr2_ref, br_ref,
               WamT_ref, wat_ref, ba_ref,
               WeAddT_ref, beadd_ref,
               Wi1_ref, Wi2_ref, WhhT_ref, blstm_ref,
               wp_ref, bp_ref,
               out_ref, value_ref, hs_ref, cwT_ref, gxm_ref, gxl_ref, t1_ref):
    mkT = mkT_ref[...]
    Wr2 = Wr2_ref[...]
    br = br_ref[...]
    Wi2 = Wi2_ref[...]
    blstm = blstm_ref[...]
    _CH = 1600

    def pro(i, _):
        sl = pl.ds(i * _CH, _CH)
        q = qe_ref[sl, :]
        logits = jnp.dot(q, mkT, preferred_element_type=jnp.float32)
        mx = jnp.max(logits, axis=1, keepdims=True)
        ex = jnp.exp(logits - mx)
        rs = 1.0 / jnp.sum(ex, axis=1, keepdims=True)
        cw = (ex * rs).astype(jnp.bfloat16)
        cwT_ref[sl, :] = jnp.swapaxes(
            cw.reshape(_CH // _B, _B, _M), 1, 2).reshape(_CH, _B)
        gxm_ref[sl, :] = jnp.dot(q, Wr2,
                                 preferred_element_type=jnp.float32) + br
        gxl_ref[sl, :] = jnp.dot(q, Wi2,
                                 preferred_element_type=jnp.float32) + blstm
        return 0

    lax.fori_loop(0, (_S * _B) // _CH, pro, 0)
    value_ref[...] = jnp.broadcast_to(
        vinit_ref[...].astype(jnp.bfloat16)[:, None, :], (_M, _B, _DV))

    Wr1 = Wr1_ref[...]
    WamT = WamT_ref[...]
    wat = wat_ref[...]
    ba = ba_ref[...]
    WeAddT = WeAddT_ref[...]
    beadd = beadd_ref[...]
    Wi1 = Wi1_ref[...]
    WhhT = WhhT_ref[...]

    def _treesum(prod):
        p8 = prod.reshape(_M // 8, 8, _B, _DV)
        sb = ((p8[:, 0] + p8[:, 1]) + (p8[:, 2] + p8[:, 3])) \
            + ((p8[:, 4] + p8[:, 5]) + (p8[:, 6] + p8[:, 7]))
        return jnp.sum(sb.astype(jnp.float32), axis=0)

    t1_ref[...] = cwT_ref[pl.ds(0, _M), :][:, :, None] * value_ref[...]
    read0 = _treesum(t1_ref[...])

    def step(t, carry):
        read, h, c = carry
        rb = read.astype(jnp.bfloat16)
        mastery = jnp.tanh(
            jnp.dot(rb, Wr1, preferred_element_type=jnp.float32)
            + gxm_ref[pl.ds(t * _B, _B), :])
        qa = jnp.tanh(
            jnp.dot(mastery.astype(jnp.bfloat16), WamT,
                    preferred_element_type=jnp.float32)
            + tcol_ref[pl.ds(t * _B, _B), :] * wat + ba)
        ea = (jnp.dot(qa.astype(jnp.bfloat16), WeAddT,
                      preferred_element_type=jnp.float32) + beadd)
        erase = jax.nn.sigmoid(ea[:, :_DV])
        addv = jnp.tanh(ea[:, _DV:])
        gates = (jnp.dot(rb, Wi1, preferred_element_type=jnp.float32)
                 + jnp.dot(h.astype(jnp.bfloat16), WhhT,
                           preferred_element_type=jnp.float32)
                 + gxl_ref[pl.ds(t * _B, _B), :])
        ig = jax.nn.sigmoid(gates[:, 0:_H])
        fg = jax.nn.sigmoid(gates[:, _H:2 * _H])
        gg = jnp.tanh(gates[:, 2 * _H:3 * _H])
        og = jax.nn.sigmoid(gates[:, 3 * _H:4 * _H])
        c = fg * c + ig * gg
        h = og * jnp.tanh(c)
        hs_ref[t] = h
        cw3 = cwT_ref[pl.ds(t * _M, _M), :][:, :, None]
        tn = jnp.minimum(t + 1, _S - 1)
        cwn = cwT_ref[pl.ds(tn * _M, _M), :][:, :, None]
        eb = erase.astype(jnp.bfloat16)
        ab = addv.astype(jnp.bfloat16)
        value = value_ref[...]
        t1 = t1_ref[...]
        nv = value - t1 * eb[None, :, :] + cw3 * ab[None, :, :]
        value_ref[...] = nv
        t1_ref[...] = cwn * nv
        read_n = _treesum(t1_ref[...])
        return read_n, h, c

    z = jnp.zeros((_B, _H), jnp.float32)
    lax.fori_loop(0, _S, step, (read0, z, z))
    hs = hs_ref[...]
    p = jnp.sum(hs * wp_ref[...][None], axis=2) + bp_ref[...]
    out_ref[...] = jax.nn.sigmoid(p)


def _scan_tc(qe, tcol, mkT, vinit, Wr1, Wr2, br, WamT, wat, ba,
             WeAddT, beadd, Wi1, Wi2, WhhT, blstm, wp, bp):
    return pl.pallas_call(
        _scan_body,
        out_shape=jax.ShapeDtypeStruct((_S, _B), jnp.float32),
        scratch_shapes=[
            pltpu.VMEM((_M, _B, _DV), jnp.bfloat16),
            pltpu.VMEM((_S, _B, _H), jnp.float32),
            pltpu.VMEM((_S * _B, _M), jnp.bfloat16),
            pltpu.VMEM((_S * _B, _FCP), jnp.float32),
            pltpu.VMEM((_S * _B, 4 * _H), jnp.float32),
            pltpu.VMEM((_M, _B, _DV), jnp.bfloat16),
        ],
    )(qe, tcol, mkT, vinit, Wr1, Wr2, br, WamT, wat, ba,
      WeAddT, beadd, Wi1, Wi2, WhhT, blstm, wp, bp)


def kernel(q_data, target, q_embed_table, mem_key, mem_value_init,
           W_read, b_read, W_a, b_a, W_e, b_e, W_add, b_add,
           W_ih, W_hh, b_ih, b_hh, W_pred, b_pred):
    idx = q_data.T.reshape(-1).astype(jnp.int32)
    qe = _sc_gather(q_embed_table, idx)
    tcol = target.T.reshape(_S * _B, 1)

    WrT = W_read.T
    pad = ((0, 0), (0, _FCP - _FC))
    Wr1 = jnp.pad(WrT[:_DV], pad)
    Wr2 = jnp.pad(WrT[_DV:], pad)
    br = jnp.pad(b_read, (0, _FCP - _FC))[None]
    WamT = jnp.pad(W_a[:, :_FC].T, ((0, _FCP - _FC), (0, 0)))
    wat = W_a[:, _FC][None]
    ba = b_a[None]
    WeAddT = jnp.concatenate([W_e.T, W_add.T], axis=1)
    beadd = jnp.concatenate([b_e, b_add])[None]
    WiT = W_ih.T
    Wi1 = WiT[:_DV]
    Wi2 = WiT[_DV:]
    WhhT = W_hh.T
    blstm = (b_ih + b_hh)[None]
    wp = W_pred
    bp = b_pred.reshape(1, 1)

    bh = jnp.bfloat16
    out = _scan_tc(qe, tcol, mkT=mem_key.T, vinit=mem_value_init,
                   Wr1=Wr1.astype(bh), Wr2=Wr2, br=br,
                   WamT=WamT.astype(bh), wat=wat, ba=ba,
                   WeAddT=WeAddT.astype(bh), beadd=beadd,
                   Wi1=Wi1.astype(bh), Wi2=Wi2, WhhT=WhhT.astype(bh),
                   blstm=blstm, wp=wp, bp=bp)
    return out.T
